# baseline (speedup 1.0000x reference)
import jax
import jax.numpy as jnp
from jax import lax
from jax.experimental import pallas as pl
from jax.experimental.pallas import tpu as pltpu
from jax.experimental.pallas import tpu_sc as plsc

_N = 10000
_D = 128
_C = 40
_NP = 10112
_PADROW = 10008
_E = 320000
_NW = 32
_K = 128
_CH = 80
_CHI = 82
_RPT = _NP // 16
_RB = 1024


def _sc_agg_body(table, src_r, dst_r, zrows, part,
                 src_v, dst_v, rows_v, acc_sh, gsem, ssem):
    c = lax.axis_index("c")
    s = lax.axis_index("s")
    wid = c * 16 + s
    base = s * _RPT
    pltpu.sync_copy(zrows.at[pl.ds(base, _RPT)], acc_sh.at[pl.ds(base, _RPT)])
    pltpu.sync_copy(src_r.at[wid], src_v)
    pltpu.sync_copy(dst_r.at[wid], dst_v)
    plsc.subcore_barrier()

    def step(j, carry):
        pltpu.async_copy(table.at[src_v.at[j]], rows_v, gsem).wait()
        pltpu.async_copy(rows_v, acc_sh.at[dst_v.at[j]], ssem, add=True).wait()
        return carry

    lax.fori_loop(0, _CH, step, 0)
    plsc.subcore_barrier()
    pltpu.sync_copy(acc_sh.at[pl.ds(base, _RPT)], part.at[c, pl.ds(base, _RPT)])


def _sc_deg_body(dst_r, zrows, ones_hbm, degpart, dst_v, ones_v, deg_sh, sem):
    c = lax.axis_index("c")
    s = lax.axis_index("s")
    wid = c * 16 + s
    base = s * _RPT
    pltpu.sync_copy(zrows.at[pl.ds(base, _RPT)], deg_sh.at[pl.ds(base, _RPT)])
    pltpu.sync_copy(ones_hbm, ones_v)
    pltpu.sync_copy(dst_r.at[wid], dst_v)
    plsc.subcore_barrier()

    def start_scatter(j):
        pltpu.async_copy(ones_v, deg_sh.at[dst_v.at[j]], sem, add=True)

    def wait_one():
        pltpu.make_async_copy(ones_v, deg_sh.at[dst_v.at[0]], sem).wait()

    for b in range(4):
        start_scatter(b)

    def step(j, carry):
        wait_one()
        start_scatter(j + 4)
        return carry

    lax.fori_loop(0, _CH - 4, step, 0)
    for b in range(4):
        wait_one()
    plsc.subcore_barrier()
    pltpu.sync_copy(deg_sh.at[pl.ds(base, _RPT)], degpart.at[c, pl.ds(base, _RPT)])


_sc_mesh = plsc.VectorSubcoreMesh(core_axis_name="c", subcore_axis_name="s")

_sc_agg = pl.kernel(
    _sc_agg_body,
    out_type=jax.ShapeDtypeStruct((2, _NP, _D), jnp.float32),
    mesh=_sc_mesh,
    scratch_types=[
        pltpu.VMEM((_CH, _K), jnp.int32),
        pltpu.VMEM((_CH, _K), jnp.int32),
        pltpu.VMEM((_K, _D), jnp.float32),
        pltpu.VMEM_SHARED((_NP, _D), jnp.float32),
        pltpu.SemaphoreType.DMA,
        pltpu.SemaphoreType.DMA,
    ],
)

_sc_deg = pl.kernel(
    _sc_deg_body,
    out_type=jax.ShapeDtypeStruct((2, _NP, _D), jnp.float32),
    mesh=_sc_mesh,
    scratch_types=[
        pltpu.VMEM((_CH, _K), jnp.int32),
        pltpu.VMEM((_K, _D), jnp.float32),
        pltpu.VMEM_SHARED((_NP, _D), jnp.float32),
        pltpu.SemaphoreType.DMA,
    ],
)


def _mm(a, b):
    return jnp.dot(a, b, preferred_element_type=jnp.float32)


def _dense1_body(part, degpart, xa, wl, wr, b, out):
    p = part[...]
    agg = p[0] + p[1]
    d = degpart[...]
    deg = d[0, :, 0:1] + d[1, :, 0:1]
    mean = agg / jnp.maximum(deg, 1.0)
    h = _mm(mean, wl[...]) + _mm(xa[...], wr[...]) + b[...]
    out[...] = jnp.maximum(h, 0.0)


_dense1 = pl.pallas_call(
    _dense1_body,
    grid=(10,),
    in_specs=[
        pl.BlockSpec((2, _RB, _D), lambda i: (0, i, 0)),
        pl.BlockSpec((2, _RB, _D), lambda i: (0, i, 0)),
        pl.BlockSpec((_RB, _D), lambda i: (i, 0)),
        pl.BlockSpec((_D, _D), lambda i: (0, 0)),
        pl.BlockSpec((_D, _D), lambda i: (0, 0)),
        pl.BlockSpec((1, _D), lambda i: (0, 0)),
    ],
    out_specs=pl.BlockSpec((_RB, _D), lambda i: (i, 0)),
    out_shape=jax.ShapeDtypeStruct((_NP, _D), jnp.float32),
)


def _dense2_body(part, degpart, h1a, w2l, w2r, b2, wc1, bc1, wc2, bc2, out):
    p = part[...]
    agg = p[0] + p[1]
    d = degpart[...]
    deg = d[0, :, 0:1] + d[1, :, 0:1]
    mean = agg / jnp.maximum(deg, 1.0)
    h2 = _mm(mean, w2l[...]) + _mm(h1a[...], w2r[...]) + b2[...]
    h2 = jnp.maximum(h2, 0.0)
    z = _mm(h2, wc1[...]) + bc1[...]
    z = _mm(z, wc2[...]) + bc2[...]
    m = jnp.max(z, axis=1, keepdims=True)
    ez = jnp.exp(z - m)
    out[...] = z - m - jnp.log(jnp.sum(ez, axis=1, keepdims=True))


_dense2 = pl.pallas_call(
    _dense2_body,
    grid=(10,),
    in_specs=[
        pl.BlockSpec((2, _RB, _D), lambda i: (0, i, 0)),
        pl.BlockSpec((2, _RB, _D), lambda i: (0, i, 0)),
        pl.BlockSpec((_RB, _D), lambda i: (i, 0)),
        pl.BlockSpec((_D, _D), lambda i: (0, 0)),
        pl.BlockSpec((_D, _D), lambda i: (0, 0)),
        pl.BlockSpec((1, _D), lambda i: (0, 0)),
        pl.BlockSpec((_D, _D), lambda i: (0, 0)),
        pl.BlockSpec((1, _D), lambda i: (0, 0)),
        pl.BlockSpec((_D, _C), lambda i: (0, 0)),
        pl.BlockSpec((1, _C), lambda i: (0, 0)),
    ],
    out_specs=pl.BlockSpec((_RB, _C), lambda i: (i, 0)),
    out_shape=jax.ShapeDtypeStruct((_N, _C), jnp.float32),
)


def kernel(x, edge_index, W1l, b1l, W1r, W2l, b2l, W2r, Wc1, bc1, Wc2, bc2):
    f32 = jnp.float32
    i32 = jnp.int32
    src = edge_index[0]
    dst = edge_index[1]
    pad = _NW * _CH * _K - _E
    src_r = jnp.concatenate([src, jnp.zeros((pad,), i32)]).reshape(_NW, _CH, _K)
    dst_r = jnp.concatenate([dst, jnp.full((pad,), _PADROW, i32)]).reshape(_NW, _CH, _K)
    xa = jnp.zeros((_NP, _D), f32).at[:_N].set(x)
    zrows = jnp.zeros((_NP, _D), f32)
    ones = jnp.ones((_K, _D), f32)

    degpart = _sc_deg(dst_r, zrows, ones)
    part1 = _sc_agg(xa, src_r, dst_r, zrows)
    h1a = _dense1(part1, degpart, xa, W1l.T, W1r.T, b1l.reshape(1, -1))
    part2 = _sc_agg(h1a, src_r, dst_r, zrows)
    out = _dense2(part2, degpart, h1a, W2l.T, W2r.T, b2l.reshape(1, -1),
                  Wc1.T, bc1.reshape(1, -1), Wc2.T, bc2.reshape(1, -1))
    return out


# speedup vs baseline: 2.1693x; 2.1693x over previous
import jax
import jax.numpy as jnp
from jax import lax
from jax.experimental import pallas as pl
from jax.experimental.pallas import tpu as pltpu
from jax.experimental.pallas import tpu_sc as plsc

_N = 10000
_D = 128
_C = 40
_NP = 10240
_E = 320000
_NW = 32
_K = 128
_CH = 79
_RPT = _NP // 16
_RB = 1024


def _sc_agg_body(table, src_r, dst_r, zrows, part, src_v, dst_v, rows_v, acc_sh):
    c = lax.axis_index("c")
    s = lax.axis_index("s")
    wid = c * 16 + s
    base = s * _RPT
    pltpu.sync_copy(zrows.at[pl.ds(base, _RPT)], acc_sh.at[pl.ds(base, _RPT)])
    pltpu.sync_copy(src_r.at[wid], src_v)
    pltpu.sync_copy(dst_r.at[wid], dst_v)
    plsc.subcore_barrier()

    def step(j, carry):
        pltpu.sync_copy(table.at[src_v.at[j]], rows_v)
        pltpu.sync_copy(rows_v, acc_sh.at[dst_v.at[j]], add=True)
        return carry

    lax.fori_loop(0, _CH, step, 0)
    plsc.subcore_barrier()
    pltpu.sync_copy(acc_sh.at[pl.ds(base, _RPT)], part.at[c, pl.ds(base, _RPT)])


def _sc_deg_body(dst_r, zrows, ones_hbm, degpart, dst_v, ones_v, deg_sh):
    c = lax.axis_index("c")
    s = lax.axis_index("s")
    wid = c * 16 + s
    base = s * _RPT
    pltpu.sync_copy(zrows.at[pl.ds(base, _RPT)], deg_sh.at[pl.ds(base, _RPT)])
    pltpu.sync_copy(ones_hbm, ones_v)
    pltpu.sync_copy(dst_r.at[wid], dst_v)
    plsc.subcore_barrier()

    def step(j, carry):
        pltpu.sync_copy(ones_v, deg_sh.at[dst_v.at[j]], add=True)
        return carry

    lax.fori_loop(0, _CH, step, 0)
    plsc.subcore_barrier()
    pltpu.sync_copy(deg_sh.at[pl.ds(base, _RPT)], degpart.at[c, pl.ds(base, _RPT)])


_sc_mesh = plsc.VectorSubcoreMesh(core_axis_name="c", subcore_axis_name="s")

_sc_agg = pl.kernel(
    _sc_agg_body,
    out_type=jax.ShapeDtypeStruct((2, _NP, _D), jnp.float32),
    mesh=_sc_mesh,
    scratch_types=[
        pltpu.VMEM((_CH, _K), jnp.int32),
        pltpu.VMEM((_CH, _K), jnp.int32),
        pltpu.VMEM((_K, _D), jnp.float32),
        pltpu.VMEM_SHARED((_NP, _D), jnp.float32),
    ],
)

_sc_deg = pl.kernel(
    _sc_deg_body,
    out_type=jax.ShapeDtypeStruct((2, _NP, _D), jnp.float32),
    mesh=_sc_mesh,
    scratch_types=[
        pltpu.VMEM((_CH, _K), jnp.int32),
        pltpu.VMEM((_K, _D), jnp.float32),
        pltpu.VMEM_SHARED((_NP, _D), jnp.float32),
    ],
)


def _mm(a, b):
    return jnp.dot(a, b, preferred_element_type=jnp.float32)


def _dense1_body(part, degpart, xa, wl, wr, b, out):
    p = part[...]
    agg = p[0] + p[1]
    d = degpart[...]
    deg = d[0, :, 0:1] + d[1, :, 0:1]
    mean = agg / jnp.maximum(deg, 1.0)
    h = _mm(mean, wl[...]) + _mm(xa[...], wr[...]) + b[...]
    out[...] = jnp.maximum(h, 0.0)


_dense1 = pl.pallas_call(
    _dense1_body,
    grid=(_NP // _RB,),
    in_specs=[
        pl.BlockSpec((2, _RB, _D), lambda i: (0, i, 0)),
        pl.BlockSpec((2, _RB, _D), lambda i: (0, i, 0)),
        pl.BlockSpec((_RB, _D), lambda i: (i, 0)),
        pl.BlockSpec((_D, _D), lambda i: (0, 0)),
        pl.BlockSpec((_D, _D), lambda i: (0, 0)),
        pl.BlockSpec((1, _D), lambda i: (0, 0)),
    ],
    out_specs=pl.BlockSpec((_RB, _D), lambda i: (i, 0)),
    out_shape=jax.ShapeDtypeStruct((_NP, _D), jnp.float32),
)


def _dense2_body(part, degpart, h1a, w2l, w2r, b2, wc1, bc1, wc2, bc2, out):
    p = part[...]
    agg = p[0] + p[1]
    d = degpart[...]
    deg = d[0, :, 0:1] + d[1, :, 0:1]
    mean = agg / jnp.maximum(deg, 1.0)
    h2 = _mm(mean, w2l[...]) + _mm(h1a[...], w2r[...]) + b2[...]
    h2 = jnp.maximum(h2, 0.0)
    z = _mm(h2, wc1[...]) + bc1[...]
    z = _mm(z, wc2[...]) + bc2[...]
    m = jnp.max(z, axis=1, keepdims=True)
    ez = jnp.exp(z - m)
    out[...] = z - m - jnp.log(jnp.sum(ez, axis=1, keepdims=True))


_dense2 = pl.pallas_call(
    _dense2_body,
    grid=(_NP // _RB,),
    in_specs=[
        pl.BlockSpec((2, _RB, _D), lambda i: (0, i, 0)),
        pl.BlockSpec((2, _RB, _D), lambda i: (0, i, 0)),
        pl.BlockSpec((_RB, _D), lambda i: (i, 0)),
        pl.BlockSpec((_D, _D), lambda i: (0, 0)),
        pl.BlockSpec((_D, _D), lambda i: (0, 0)),
        pl.BlockSpec((1, _D), lambda i: (0, 0)),
        pl.BlockSpec((_D, _D), lambda i: (0, 0)),
        pl.BlockSpec((1, _D), lambda i: (0, 0)),
        pl.BlockSpec((_D, _C), lambda i: (0, 0)),
        pl.BlockSpec((1, _C), lambda i: (0, 0)),
    ],
    out_specs=pl.BlockSpec((_RB, _C), lambda i: (i, 0)),
    out_shape=jax.ShapeDtypeStruct((_N, _C), jnp.float32),
)


def kernel(x, edge_index, W1l, b1l, W1r, W2l, b2l, W2r, Wc1, bc1, Wc2, bc2):
    f32 = jnp.float32
    src = edge_index[0]
    dst = edge_index[1]
    pad = _NW * _CH * _K - _E
    src_r = jnp.concatenate([src, jnp.zeros((pad,), jnp.int32)]).reshape(_NW, _CH, _K)
    dst_r = jnp.concatenate([dst, jnp.full((pad,), _N + 100, jnp.int32)]).reshape(_NW, _CH, _K)
    xa = jnp.zeros((_NP, _D), f32).at[:_N].set(x)
    zrows = jnp.zeros((_NP, _D), f32)
    ones = jnp.ones((_K, _D), f32)

    degpart = _sc_deg(dst_r, zrows, ones)
    part1 = _sc_agg(xa, src_r, dst_r, zrows)
    h1a = _dense1(part1, degpart, xa, W1l.T, W1r.T, b1l.reshape(1, -1))
    part2 = _sc_agg(h1a, src_r, dst_r, zrows)
    out = _dense2(part2, degpart, h1a, W2l.T, W2r.T, b2l.reshape(1, -1),
                  Wc1.T, bc1.reshape(1, -1), Wc2.T, bc2.reshape(1, -1))
    return out
